# baseline (speedup 1.0000x reference)
import functools

import jax
import jax.numpy as jnp
from jax import lax
from jax.experimental import pallas as pl
from jax.experimental.pallas import tpu as pltpu
from jax.experimental.pallas import tpu_sc as plsc

_MOMENTUM = 0.9
_B, _N, _D, _M = 1024, 77, 768, 100000
_NC, _NS, _L = 2, 16, 16
_NW = _NC * _NS
_BPW = _B // _NW

_mesh = plsc.VectorSubcoreMesh(
    core_axis_name="c", subcore_axis_name="s", num_cores=_NC, num_subcores=_NS
)


@functools.partial(
    pl.kernel,
    out_type=(),
    mesh=_mesh,
    scratch_types=[
        pltpu.VMEM((_BPW,), jnp.int32),
        pltpu.VMEM((_BPW,), jnp.int32),
        pltpu.VMEM((_BPW,), jnp.int32),
        pltpu.VMEM((_BPW, _D), jnp.float32),
        pltpu.VMEM((_BPW, _D), jnp.float32),
        pltpu.SemaphoreType.DMA,
    ],
)
def _sc_update(meta_hbm, ids_hbm, pos_hbm, emb_ref, mem_ref,
               idx_v, pos_v, eidx_v, rows_v, meta_v, sem):
    wid = lax.axis_index("s") * _NC + lax.axis_index("c")
    base = wid * _BPW

    pltpu.sync_copy(ids_hbm.at[pl.ds(base, _BPW)], idx_v)
    pltpu.sync_copy(pos_hbm.at[pl.ds(base, _BPW)], pos_v)
    pltpu.sync_copy(meta_hbm.at[pl.ds(base, _BPW)], meta_v)

    pltpu.async_copy(mem_ref.at[idx_v], rows_v, sem).wait()

    plsc.subcore_barrier()

    def _row(r, carry):
        for c in range(_D // _L):
            s = pl.ds(c * _L, _L)
            rows_v[r, s] = (
                rows_v[r, s] * _MOMENTUM + meta_v[r, s] * (1.0 - _MOMENTUM)
            )
        return carry

    lax.fori_loop(0, _BPW, _row, 0)

    for c in range(_BPW // _L):
        s = pl.ds(c * _L, _L)
        row_id = base + c * _L + lax.iota(jnp.int32, _L)
        eidx_v[s] = row_id * _N + pos_v[s]

    upd_mem = pltpu.async_copy(rows_v, mem_ref.at[idx_v], sem)
    upd_emb = pltpu.async_copy(rows_v, emb_ref.at[eidx_v], sem)
    upd_mem.wait()
    upd_emb.wait()


def kernel(embedded_text, meta, mem, ids, pos):
    emb_ref = jax.new_ref(embedded_text.reshape(_B * _N, _D))
    mem_ref = jax.new_ref(mem)
    _sc_update(meta, ids.astype(jnp.int32), pos.astype(jnp.int32),
               emb_ref, mem_ref)
    return emb_ref[...].reshape(_B, _N, _D), mem_ref[...]


# speedup vs baseline: 1.0003x; 1.0003x over previous
import functools

import jax
import jax.numpy as jnp
from jax import lax
from jax.experimental import pallas as pl
from jax.experimental.pallas import tpu as pltpu
from jax.experimental.pallas import tpu_sc as plsc

_MOMENTUM = 0.9
_B, _N, _D, _M = 1024, 77, 768, 100000
_NC, _NS, _L = 2, 16, 16
_NW = _NC * _NS
_BPW = _B // _NW

_mesh = plsc.VectorSubcoreMesh(
    core_axis_name="c", subcore_axis_name="s", num_cores=_NC, num_subcores=_NS
)


@functools.partial(
    pl.kernel,
    out_type=(),
    mesh=_mesh,
    scratch_types=[
        pltpu.VMEM((_BPW,), jnp.int32),
        pltpu.VMEM((_BPW,), jnp.int32),
        pltpu.VMEM((_BPW,), jnp.int32),
        pltpu.VMEM((_BPW, _D), jnp.float32),
        pltpu.VMEM((_BPW, _D), jnp.float32),
        pltpu.SemaphoreType.DMA,
    ],
)
def _sc_update(meta_hbm, ids_hbm, pos_hbm, emb_ref, mem_ref,
               idx_v, pos_v, eidx_v, rows_v, meta_v, sem):
    wid = lax.axis_index("s") * _NC + lax.axis_index("c")
    base = wid * _BPW

    pltpu.sync_copy(ids_hbm.at[pl.ds(base, _BPW)], idx_v)
    pltpu.sync_copy(pos_hbm.at[pl.ds(base, _BPW)], pos_v)
    pltpu.sync_copy(meta_hbm.at[pl.ds(base, _BPW)], meta_v)

    pltpu.async_copy(mem_ref.at[idx_v], rows_v, sem).wait()

    plsc.subcore_barrier()

    def _row(r, carry):
        for c in range(_D // _L):
            s = pl.ds(c * _L, _L)
            rows_v[r, s] = (
                rows_v[r, s] * _MOMENTUM + meta_v[r, s] * (1.0 - _MOMENTUM)
            )
        return carry

    lax.fori_loop(0, _BPW, _row, 0)

    for c in range(_BPW // _L):
        s = pl.ds(c * _L, _L)
        row_id = base + c * _L + lax.iota(jnp.int32, _L)
        eidx_v[s] = row_id * _N + pos_v[s]

    upd_mem = pltpu.async_copy(rows_v, mem_ref.at[idx_v], sem)
    upd_emb = pltpu.async_copy(rows_v, emb_ref.at[eidx_v], sem)
    upd_mem.wait()
    upd_emb.wait()


def kernel(embedded_text, meta, mem, ids, pos):
    emb_ref = jax.new_ref(embedded_text.reshape(_B * _N, _D))
    mem_ref = jax.new_ref(mem)
    _sc_update(meta, ids.astype(jnp.int32), pos.astype(jnp.int32),
               emb_ref, mem_ref)
    return jax.freeze(emb_ref).reshape(_B, _N, _D), jax.freeze(mem_ref)


# speedup vs baseline: 2.4967x; 2.4958x over previous
import functools

import jax
import jax.numpy as jnp
from jax import lax
from jax.experimental import pallas as pl
from jax.experimental.pallas import tpu as pltpu
from jax.experimental.pallas import tpu_sc as plsc

_MOMENTUM = 0.9
_B, _N, _D, _M = 1024, 77, 768, 100000
_NC, _NS, _L = 2, 16, 16
_NW = _NC * _NS
_BPW = _B // _NW

_mesh = plsc.VectorSubcoreMesh(
    core_axis_name="c", subcore_axis_name="s", num_cores=_NC, num_subcores=_NS
)


@functools.partial(
    pl.kernel,
    out_type=(),
    mesh=_mesh,
    scratch_types=[
        pltpu.VMEM((_BPW,), jnp.int32),
        pltpu.VMEM((_BPW,), jnp.int32),
        pltpu.VMEM((_BPW,), jnp.int32),
        pltpu.VMEM((_BPW, _D), jnp.float32),
        pltpu.VMEM((_BPW, _D), jnp.float32),
        pltpu.SemaphoreType.DMA,
    ],
)
def _sc_update(meta_hbm, ids_hbm, pos_hbm, emb_ref, mem_ref,
               idx_v, pos_v, eidx_v, rows_v, meta_v, sem):
    wid = lax.axis_index("s") * _NC + lax.axis_index("c")
    base = wid * _BPW

    pltpu.sync_copy(ids_hbm.at[pl.ds(base, _BPW)], idx_v)
    pltpu.sync_copy(pos_hbm.at[pl.ds(base, _BPW)], pos_v)
    pltpu.sync_copy(meta_hbm.at[pl.ds(base, _BPW)], meta_v)

    pltpu.async_copy(mem_ref.at[idx_v], rows_v, sem).wait()

    plsc.subcore_barrier()

    def _row(r, carry):
        for c in range(_D // _L):
            s = pl.ds(c * _L, _L)
            rows_v[r, s] = (
                rows_v[r, s] * _MOMENTUM + meta_v[r, s] * (1.0 - _MOMENTUM)
            )
        return carry

    lax.fori_loop(0, _BPW, _row, 0)

    for c in range(_BPW // _L):
        s = pl.ds(c * _L, _L)
        row_id = base + c * _L + lax.iota(jnp.int32, _L)
        eidx_v[s] = pos_v[s] * _B + row_id

    upd_mem = pltpu.async_copy(rows_v, mem_ref.at[idx_v], sem)
    upd_emb = pltpu.async_copy(rows_v, emb_ref.at[eidx_v], sem)
    upd_mem.wait()
    upd_emb.wait()


def kernel(embedded_text, meta, mem, ids, pos):
    emb_ref = jax.new_ref(jnp.swapaxes(embedded_text, 0, 1).reshape(_N * _B, _D))
    mem_ref = jax.new_ref(mem)
    _sc_update(meta, ids.astype(jnp.int32), pos.astype(jnp.int32),
               emb_ref, mem_ref)
    new_emb = jnp.swapaxes(jax.freeze(emb_ref).reshape(_N, _B, _D), 0, 1)
    return new_emb, jax.freeze(mem_ref)
